# baseline (speedup 1.0000x reference)
import functools

import numpy as np

import jax
import jax.numpy as jnp
from jax import lax
from jax.experimental import pallas as pl
from jax.experimental.pallas import tpu as pltpu
from jax.experimental.pallas import tpu_sc as plsc

_C = 32
_K = 3
_PS = 4
_H = 32
_W = 32
_PN = _H // _PS
_NP = _PN * _PN
_PP = _PS * _PS
_PIX = _H * _W
_CU = _PP * _C
_BIG = 3.4e38

_NC = 2
_NS = 16
_NW = _NC * _NS
_GQ = 16
_NG = _NP // _GQ
_BB = 5


def _extraction_matrices():
    e = np.zeros((_PIX, _PP * _NP), dtype=np.float32)
    for psh in range(_PS):
        for psw in range(_PS):
            pp = psh * _PS + psw
            for hr in range(_PN):
                for wr in range(_PN):
                    q = hr * _PN + wr
                    pix = (_PS * hr + psh) * _W + _PS * wr + psw
                    e[pix, pp * _NP + q] = 1.0
    e2 = e.reshape(_PIX, _PP, _NP).transpose(1, 2, 0).copy()
    return e, e2


_E_NP, _E2_NP = _extraction_matrices()


def _leaky(v):
    return jnp.where(v >= 0, v, 0.1 * v)


def _extract(lf, e_ref):
    return lax.dot_general(lf, e_ref[...], (((1,), (0,)), ((), ())),
                           preferred_element_type=jnp.float32)


def _dist_kernel(lf_ref, e_ref, dt_ref):
    for bi in range(_BB):
        _dist_one(lf_ref, e_ref, dt_ref, bi)


def _dist_one(lf_ref, e_ref, dt_ref, bi):
    a = _extract(lf_ref[bi], e_ref)
    aa = a * a
    ones_r = jnp.ones((1, _C), dtype=jnp.float32)
    gp = [jnp.zeros((_NP, _NP), dtype=jnp.float32) for _ in range(4)]
    for pp in range(_PP):
        ap = a[:, pp * _NP:(pp + 1) * _NP]
        gp[pp % 4] = gp[pp % 4] + lax.dot_general(
            ap, ap, (((0,), (0,)), ((), ())),
            preferred_element_type=jnp.float32)
    g = (gp[0] + gp[1]) + (gp[2] + gp[3])
    s = lax.dot_general(ones_r, aa, (((1,), (0,)), ((), ())),
                        preferred_element_type=jnp.float32)
    nrow = jnp.zeros((1, _NP), dtype=jnp.float32)
    for pp in range(_PP):
        nrow = nrow + s[:, pp * _NP:(pp + 1) * _NP]
    ncol = jnp.transpose(nrow)
    d = ncol + nrow - 2.0 * g
    dtfull = jnp.concatenate(
        [d, jnp.broadcast_to(nrow, (_NP, _NP))], axis=0)
    for grp in range(_NG):
        dt_ref[bi, grp] = dtfull[:, grp * _GQ:(grp + 1) * _GQ]


def _sc_body(num_groups, dt_hbm, idx_hbm, dloc, ibuf, sem):
    wid = lax.axis_index("s") * _NC + lax.axis_index("c")
    zero_i = jnp.zeros((_GQ,), dtype=jnp.int32)
    big_f = jnp.full((_GQ,), _BIG, dtype=jnp.float32)
    for t in range((num_groups + _NW - 1) // _NW):
        gid = t * _NW + wid

        @pl.when(gid < num_groups)
        def _():
            b = gid // _NG
            grp = gid % _NG
            pltpu.sync_copy(dt_hbm.at[b, grp], dloc)

            def step(j, carry):
                m1, m2, m3, i1, i2, i3 = carry
                v = dloc[j]
                jv = jnp.full((_GQ,), j, dtype=jnp.int32)
                c1 = v < m1
                c2 = v < m2
                c3 = v < m3
                m3n = jnp.where(c3, jnp.where(c2, m2, v), m3)
                i3n = jnp.where(c3, jnp.where(c2, i2, jv), i3)
                m2n = jnp.where(c2, jnp.where(c1, m1, v), m2)
                i2n = jnp.where(c2, jnp.where(c1, i1, jv), i2)
                m1n = jnp.where(c1, v, m1)
                i1n = jnp.where(c1, jv, i1)
                return m1n, m2n, m3n, i1n, i2n, i3n

            _, _, _, i1, i2, i3 = lax.fori_loop(
                0, 2 * _NP, step,
                (big_f, big_f, big_f, zero_i, zero_i, zero_i))
            ibuf[0] = i1
            ibuf[1] = i2
            ibuf[2] = i3
            pltpu.sync_copy(ibuf, idx_hbm.at[b, grp])


def _main_kernel(lf_ref, idx_ref, e_ref, w1_ref, w2_ref, o_ref):
    for bi in range(_BB):
        _main_one(lf_ref, idx_ref, e_ref, w1_ref, w2_ref, o_ref, bi)


def _main_one(lf_ref, idx_ref, e_ref, w1_ref, w2_ref, o_ref, bi):
    lf = lf_ref[bi]
    a = _extract(lf, e_ref)
    idx = idx_ref[bi]
    row_iota = lax.broadcasted_iota(jnp.int32, (_NP, _NP), 0)
    sel_t = []
    for k in range(_K):
        row = jnp.concatenate(
            [idx[g, k][None, :] for g in range(_NG)], axis=1)
        sel_t.append((row_iota == row).astype(jnp.float32))
    s_stack = jnp.concatenate(sel_t, axis=0)
    z_all = [lax.dot_general(a, w1_ref[k], (((0,), (1,)), ((), ())),
                             preferred_element_type=jnp.float32)
             for k in range(_K)]
    accs = []
    for pp in range(_PP):
        z_stack = jnp.concatenate(
            [z_all[k][pp * _NP:(pp + 1) * _NP] for k in range(_K)],
            axis=0)
        accs.append(lax.dot_general(s_stack, z_stack,
                                    (((0,), (0,)), ((), ())),
                                    preferred_element_type=jnp.float32))
    y_stack = _leaky(jnp.concatenate(accs, axis=0))
    yimg = lax.dot_general(y_stack, e_ref[...], (((0,), (1,)), ((), ())),
                           preferred_element_type=jnp.float32)
    z2 = jnp.concatenate([lf, yimg], axis=0)
    hi = lax.broadcasted_iota(jnp.int32, (2 * _C, _PIX), 1) // _W
    wi = lax.broadcasted_iota(jnp.int32, (2 * _C, _PIX), 1) % _W
    parts = [jnp.zeros((_C, _PIX), dtype=jnp.float32) for _ in range(3)]
    for dy in range(3):
        for dx in range(3):
            t = dy * 3 + dx
            sft = _W * (dy - 1) + (dx - 1)
            mask = None
            if dy == 0:
                mask = hi >= 1
            elif dy == 2:
                mask = hi < (_H - 1)
            if dx == 0:
                mw = wi >= 1
                mask = mw if mask is None else (mask & mw)
            elif dx == 2:
                mw = wi < (_W - 1)
                mask = mw if mask is None else (mask & mw)
            zs = z2 if sft == 0 else pltpu.roll(z2, (-sft) % _PIX, 1)
            if mask is not None:
                zs = jnp.where(mask, zs, 0.0)
            parts[t % 3] = parts[t % 3] + lax.dot_general(
                w2_ref[t], zs, (((1,), (0,)), ((), ())),
                preferred_element_type=jnp.float32)
    o_ref[bi] = _leaky(parts[0] + parts[1] + parts[2])


def _stage_dist(lf2d, e):
    B = lf2d.shape[0]
    return pl.pallas_call(
        _dist_kernel,
        grid=(B // _BB,),
        in_specs=[
            pl.BlockSpec((_BB, _C, _PIX), lambda b: (b, 0, 0)),
            pl.BlockSpec((_PIX, _PIX), lambda b: (0, 0)),
        ],
        out_specs=pl.BlockSpec((_BB, _NG, 2 * _NP, _GQ),
                               lambda b: (b, 0, 0, 0)),
        out_shape=jax.ShapeDtypeStruct((B, _NG, 2 * _NP, _GQ),
                                       jnp.float32),
    )(lf2d, e)


def _stage_sc(dt):
    B = dt.shape[0]
    num_groups = B * _NG
    mesh = plsc.VectorSubcoreMesh(core_axis_name="c", subcore_axis_name="s")
    return pl.kernel(
        functools.partial(_sc_body, num_groups),
        out_type=jax.ShapeDtypeStruct((B, _NG, _K, _GQ), jnp.int32),
        mesh=mesh,
        scratch_types=[
            pltpu.VMEM((2 * _NP, _GQ), jnp.float32),
            pltpu.VMEM((_K, _GQ), jnp.int32),
            pltpu.SemaphoreType.DMA,
        ],
    )(dt)


def _stage_main(lf2d, idx, e, w1, w2):
    B = lf2d.shape[0]
    return pl.pallas_call(
        _main_kernel,
        grid=(B // _BB,),
        in_specs=[
            pl.BlockSpec((_BB, _C, _PIX), lambda b: (b, 0, 0)),
            pl.BlockSpec((_BB, _NG, _K, _GQ), lambda b: (b, 0, 0, 0)),
            pl.BlockSpec((_PIX, _PIX), lambda b: (0, 0)),
            pl.BlockSpec((_K, _C, _C), lambda b: (0, 0, 0)),
            pl.BlockSpec((9, _C, 2 * _C), lambda b: (0, 0, 0)),
        ],
        out_specs=pl.BlockSpec((_BB, _C, _PIX), lambda b: (b, 0, 0)),
        out_shape=jax.ShapeDtypeStruct((B, _C, _PIX), jnp.float32),
    )(lf2d, idx, e, w1, w2)


@jax.jit
def kernel(lf_fea, w_agg1, w_agg2):
    B = lf_fea.shape[0]
    lf2d = lf_fea.reshape(B, _C, _PIX)
    e = jnp.asarray(_E_NP)
    w1 = w_agg1.reshape(_C, _K, _C).transpose(1, 0, 2)
    w2 = w_agg2.transpose(2, 3, 0, 1).reshape(9, _C, 2 * _C)

    split = (B // (2 * _BB)) * _BB
    chunks = [(0, split), (split, B)] if 0 < split < B else [(0, B)]
    dts = [_stage_dist(lf2d[s0:s1], e) for s0, s1 in chunks]
    idxs = [_stage_sc(dtc) for dtc in dts]
    outs = [_stage_main(lf2d[s0:s1], idxc, e, w1, w2)
            for (s0, s1), idxc in zip(chunks, idxs)]
    out = jnp.concatenate(outs, axis=0) if len(outs) > 1 else outs[0]
    return out.reshape(B, _C, _H, _W)


# speedup vs baseline: 1.0051x; 1.0051x over previous
import functools

import numpy as np

import jax
import jax.numpy as jnp
from jax import lax
from jax.experimental import pallas as pl
from jax.experimental.pallas import tpu as pltpu
from jax.experimental.pallas import tpu_sc as plsc

_C = 32
_K = 3
_PS = 4
_H = 32
_W = 32
_PN = _H // _PS
_NP = _PN * _PN
_PP = _PS * _PS
_PIX = _H * _W
_CU = _PP * _C
_BIG = 3.4e38

_NC = 2
_NS = 16
_NW = _NC * _NS
_GQ = 16
_NG = _NP // _GQ
_BB = 25


def _extraction_matrices():
    e = np.zeros((_PIX, _PP * _NP), dtype=np.float32)
    for psh in range(_PS):
        for psw in range(_PS):
            pp = psh * _PS + psw
            for hr in range(_PN):
                for wr in range(_PN):
                    q = hr * _PN + wr
                    pix = (_PS * hr + psh) * _W + _PS * wr + psw
                    e[pix, pp * _NP + q] = 1.0
    e2 = e.reshape(_PIX, _PP, _NP).transpose(1, 2, 0).copy()
    return e, e2


_E_NP, _E2_NP = _extraction_matrices()


def _leaky(v):
    return jnp.where(v >= 0, v, 0.1 * v)


def _extract(lf, e_ref):
    return lax.dot_general(lf, e_ref[...], (((1,), (0,)), ((), ())),
                           preferred_element_type=jnp.float32)


def _dist_kernel(lf_ref, e_ref, dt_ref):
    for bi in range(_BB):
        _dist_one(lf_ref, e_ref, dt_ref, bi)


def _dist_one(lf_ref, e_ref, dt_ref, bi):
    a = _extract(lf_ref[bi], e_ref)
    aa = a * a
    ones_r = jnp.ones((1, _C), dtype=jnp.float32)
    gp = [jnp.zeros((_NP, _NP), dtype=jnp.float32) for _ in range(4)]
    for pp in range(_PP):
        ap = a[:, pp * _NP:(pp + 1) * _NP]
        gp[pp % 4] = gp[pp % 4] + lax.dot_general(
            ap, ap, (((0,), (0,)), ((), ())),
            preferred_element_type=jnp.float32)
    g = (gp[0] + gp[1]) + (gp[2] + gp[3])
    s = lax.dot_general(ones_r, aa, (((1,), (0,)), ((), ())),
                        preferred_element_type=jnp.float32)
    nrow = jnp.zeros((1, _NP), dtype=jnp.float32)
    for pp in range(_PP):
        nrow = nrow + s[:, pp * _NP:(pp + 1) * _NP]
    ncol = jnp.transpose(nrow)
    d = ncol + nrow - 2.0 * g
    dtfull = jnp.concatenate(
        [d, jnp.broadcast_to(nrow, (_NP, _NP))], axis=0)
    for grp in range(_NG):
        dt_ref[bi, grp] = dtfull[:, grp * _GQ:(grp + 1) * _GQ]


def _sc_body(num_groups, dt_hbm, idx_hbm, dloc, ibuf, sem):
    wid = lax.axis_index("s") * _NC + lax.axis_index("c")
    zero_i = jnp.zeros((_GQ,), dtype=jnp.int32)
    big_f = jnp.full((_GQ,), _BIG, dtype=jnp.float32)
    for t in range((num_groups + _NW - 1) // _NW):
        gid = t * _NW + wid

        @pl.when(gid < num_groups)
        def _():
            b = gid // _NG
            grp = gid % _NG
            pltpu.sync_copy(dt_hbm.at[b, grp], dloc)

            def step(j, carry):
                m1, m2, m3, i1, i2, i3 = carry
                v = dloc[j]
                jv = jnp.full((_GQ,), j, dtype=jnp.int32)
                c1 = v < m1
                c2 = v < m2
                c3 = v < m3
                m3n = jnp.where(c3, jnp.where(c2, m2, v), m3)
                i3n = jnp.where(c3, jnp.where(c2, i2, jv), i3)
                m2n = jnp.where(c2, jnp.where(c1, m1, v), m2)
                i2n = jnp.where(c2, jnp.where(c1, i1, jv), i2)
                m1n = jnp.where(c1, v, m1)
                i1n = jnp.where(c1, jv, i1)
                return m1n, m2n, m3n, i1n, i2n, i3n

            _, _, _, i1, i2, i3 = lax.fori_loop(
                0, 2 * _NP, step,
                (big_f, big_f, big_f, zero_i, zero_i, zero_i))
            ibuf[0] = i1
            ibuf[1] = i2
            ibuf[2] = i3
            pltpu.sync_copy(ibuf, idx_hbm.at[b, grp])


def _main_kernel(lf_ref, idx_ref, e_ref, w1_ref, w2_ref, o_ref):
    for bi in range(_BB):
        _main_one(lf_ref, idx_ref, e_ref, w1_ref, w2_ref, o_ref, bi)


def _main_one(lf_ref, idx_ref, e_ref, w1_ref, w2_ref, o_ref, bi):
    lf = lf_ref[bi]
    a = _extract(lf, e_ref)
    idx = idx_ref[bi]
    row_iota = lax.broadcasted_iota(jnp.int32, (_NP, _NP), 0)
    sel_t = []
    for k in range(_K):
        row = jnp.concatenate(
            [idx[g, k][None, :] for g in range(_NG)], axis=1)
        sel_t.append((row_iota == row).astype(jnp.float32))
    s_stack = jnp.concatenate(sel_t, axis=0)
    z_all = [lax.dot_general(a, w1_ref[k], (((0,), (1,)), ((), ())),
                             preferred_element_type=jnp.float32)
             for k in range(_K)]
    accs = []
    for pp in range(_PP):
        z_stack = jnp.concatenate(
            [z_all[k][pp * _NP:(pp + 1) * _NP] for k in range(_K)],
            axis=0)
        accs.append(lax.dot_general(s_stack, z_stack,
                                    (((0,), (0,)), ((), ())),
                                    preferred_element_type=jnp.float32))
    y_stack = _leaky(jnp.concatenate(accs, axis=0))
    yimg = lax.dot_general(y_stack, e_ref[...], (((0,), (1,)), ((), ())),
                           preferred_element_type=jnp.float32)
    z2 = jnp.concatenate([lf, yimg], axis=0)
    hi = lax.broadcasted_iota(jnp.int32, (2 * _C, _PIX), 1) // _W
    wi = lax.broadcasted_iota(jnp.int32, (2 * _C, _PIX), 1) % _W
    parts = [jnp.zeros((_C, _PIX), dtype=jnp.float32) for _ in range(3)]
    for dy in range(3):
        for dx in range(3):
            t = dy * 3 + dx
            sft = _W * (dy - 1) + (dx - 1)
            mask = None
            if dy == 0:
                mask = hi >= 1
            elif dy == 2:
                mask = hi < (_H - 1)
            if dx == 0:
                mw = wi >= 1
                mask = mw if mask is None else (mask & mw)
            elif dx == 2:
                mw = wi < (_W - 1)
                mask = mw if mask is None else (mask & mw)
            zs = z2 if sft == 0 else pltpu.roll(z2, (-sft) % _PIX, 1)
            if mask is not None:
                zs = jnp.where(mask, zs, 0.0)
            parts[t % 3] = parts[t % 3] + lax.dot_general(
                w2_ref[t], zs, (((1,), (0,)), ((), ())),
                preferred_element_type=jnp.float32)
    o_ref[bi] = _leaky(parts[0] + parts[1] + parts[2])


@jax.jit
def kernel(lf_fea, w_agg1, w_agg2):
    B = lf_fea.shape[0]
    lf2d = lf_fea.reshape(B, _C, _PIX)
    e = jnp.asarray(_E_NP)
    w1 = w_agg1.reshape(_C, _K, _C).transpose(1, 0, 2)
    w2 = w_agg2.transpose(2, 3, 0, 1).reshape(9, _C, 2 * _C)

    dt = pl.pallas_call(
        _dist_kernel,
        grid=(B // _BB,),
        in_specs=[
            pl.BlockSpec((_BB, _C, _PIX), lambda b: (b, 0, 0)),
            pl.BlockSpec((_PIX, _PIX), lambda b: (0, 0)),
        ],
        out_specs=pl.BlockSpec((_BB, _NG, 2 * _NP, _GQ),
                               lambda b: (b, 0, 0, 0)),
        out_shape=jax.ShapeDtypeStruct((B, _NG, 2 * _NP, _GQ),
                                       jnp.float32),
    )(lf2d, e)

    num_groups = B * _NG
    mesh = plsc.VectorSubcoreMesh(core_axis_name="c", subcore_axis_name="s")
    idx = pl.kernel(
        functools.partial(_sc_body, num_groups),
        out_type=jax.ShapeDtypeStruct((B, _NG, _K, _GQ), jnp.int32),
        mesh=mesh,
        scratch_types=[
            pltpu.VMEM((2 * _NP, _GQ), jnp.float32),
            pltpu.VMEM((_K, _GQ), jnp.int32),
            pltpu.SemaphoreType.DMA,
        ],
    )(dt)

    out = pl.pallas_call(
        _main_kernel,
        grid=(B // _BB,),
        in_specs=[
            pl.BlockSpec((_BB, _C, _PIX), lambda b: (b, 0, 0)),
            pl.BlockSpec((_BB, _NG, _K, _GQ), lambda b: (b, 0, 0, 0)),
            pl.BlockSpec((_PIX, _PIX), lambda b: (0, 0)),
            pl.BlockSpec((_K, _C, _C), lambda b: (0, 0, 0)),
            pl.BlockSpec((9, _C, 2 * _C), lambda b: (0, 0, 0)),
        ],
        out_specs=pl.BlockSpec((_BB, _C, _PIX), lambda b: (b, 0, 0)),
        out_shape=jax.ShapeDtypeStruct((B, _C, _PIX), jnp.float32),
    )(lf2d, idx, e, w1, w2)
    return out.reshape(B, _C, _H, _W)


# speedup vs baseline: 1.0582x; 1.0529x over previous
import functools

import numpy as np

import jax
import jax.numpy as jnp
from jax import lax
from jax.experimental import pallas as pl
from jax.experimental.pallas import tpu as pltpu
from jax.experimental.pallas import tpu_sc as plsc

_C = 32
_K = 3
_PS = 4
_H = 32
_W = 32
_PN = _H // _PS
_NP = _PN * _PN
_PP = _PS * _PS
_PIX = _H * _W
_CU = _PP * _C
_BIG = 3.4e38

_NC = 2
_NS = 16
_NW = _NC * _NS
_GQ = 16
_NG = _NP // _GQ
_BB = 5


def _extraction_matrices():
    e = np.zeros((_PIX, _PP * _NP), dtype=np.float32)
    for psh in range(_PS):
        for psw in range(_PS):
            pp = psh * _PS + psw
            for hr in range(_PN):
                for wr in range(_PN):
                    q = hr * _PN + wr
                    pix = (_PS * hr + psh) * _W + _PS * wr + psw
                    e[pix, pp * _NP + q] = 1.0
    e2 = e.reshape(_PIX, _PP, _NP).transpose(1, 2, 0).copy()
    return e, e2


_E_NP, _E2_NP = _extraction_matrices()


def _leaky(v):
    return jnp.where(v >= 0, v, 0.1 * v)


def _extract(lf, e_ref):
    return lax.dot_general(lf, e_ref[...], (((1,), (0,)), ((), ())),
                           preferred_element_type=jnp.float32)


def _dist_kernel(lf_ref, e_ref, dt_ref):
    for bi in range(_BB):
        _dist_one(lf_ref, e_ref, dt_ref, bi)


def _dist_one(lf_ref, e_ref, dt_ref, bi):
    a = _extract(lf_ref[bi], e_ref)
    aa = a * a
    ones_r = jnp.ones((1, _C), dtype=jnp.float32)
    gp = [jnp.zeros((_NP, _NP), dtype=jnp.float32) for _ in range(4)]
    for pp in range(_PP):
        ap = a[:, pp * _NP:(pp + 1) * _NP]
        gp[pp % 4] = gp[pp % 4] + lax.dot_general(
            ap, ap, (((0,), (0,)), ((), ())),
            preferred_element_type=jnp.float32)
    g = (gp[0] + gp[1]) + (gp[2] + gp[3])
    s = lax.dot_general(ones_r, aa, (((1,), (0,)), ((), ())),
                        preferred_element_type=jnp.float32)
    nrow = jnp.zeros((1, _NP), dtype=jnp.float32)
    for pp in range(_PP):
        nrow = nrow + s[:, pp * _NP:(pp + 1) * _NP]
    ncol = jnp.transpose(nrow)
    d = ncol + nrow - 2.0 * g
    dtfull = jnp.concatenate(
        [d, jnp.broadcast_to(nrow, (8, _NP))], axis=0)
    for grp in range(_NG):
        dt_ref[bi, grp] = dtfull[:, grp * _GQ:(grp + 1) * _GQ]


def _sc_body(num_groups, dt_hbm, idx_hbm, dloc, ibuf, sem):
    wid = lax.axis_index("s") * _NC + lax.axis_index("c")
    zero_i = jnp.zeros((_GQ,), dtype=jnp.int32)
    big_f = jnp.full((_GQ,), _BIG, dtype=jnp.float32)
    for t in range((num_groups + _NW - 1) // _NW):
        gid = t * _NW + wid

        @pl.when(gid < num_groups)
        def _():
            b = gid // _NG
            grp = gid % _NG
            pltpu.sync_copy(dt_hbm.at[b, grp], dloc)

            def step(j, carry):
                m1, m2, m3, i1, i2, i3 = carry
                v = dloc[jnp.minimum(j, _NP)]
                jv = jnp.full((_GQ,), j, dtype=jnp.int32)
                c1 = v < m1
                c2 = v < m2
                c3 = v < m3
                m3n = jnp.where(c3, jnp.where(c2, m2, v), m3)
                i3n = jnp.where(c3, jnp.where(c2, i2, jv), i3)
                m2n = jnp.where(c2, jnp.where(c1, m1, v), m2)
                i2n = jnp.where(c2, jnp.where(c1, i1, jv), i2)
                m1n = jnp.where(c1, v, m1)
                i1n = jnp.where(c1, jv, i1)
                return m1n, m2n, m3n, i1n, i2n, i3n

            carry = lax.fori_loop(
                0, _NP, step,
                (big_f, big_f, big_f, zero_i, zero_i, zero_i))
            for _ in range(_K):
                carry = step(_NP, carry)
            _, _, _, i1, i2, i3 = carry
            ibuf[0] = i1
            ibuf[1] = i2
            ibuf[2] = i3
            pltpu.sync_copy(ibuf, idx_hbm.at[b, grp])


def _main_kernel(lf_ref, idx_ref, e_ref, w1_ref, w2_ref, o_ref):
    for bi in range(_BB):
        _main_one(lf_ref, idx_ref, e_ref, w1_ref, w2_ref, o_ref, bi)


def _main_one(lf_ref, idx_ref, e_ref, w1_ref, w2_ref, o_ref, bi):
    lf = lf_ref[bi]
    a = _extract(lf, e_ref)
    idx = idx_ref[bi]
    row_iota = lax.broadcasted_iota(jnp.int32, (_NP, _NP), 0)
    sel_t = []
    for k in range(_K):
        row = jnp.concatenate(
            [idx[g, k][None, :] for g in range(_NG)], axis=1)
        sel_t.append((row_iota == row).astype(jnp.float32))
    s_stack = jnp.concatenate(sel_t, axis=0)
    z_all = [lax.dot_general(a, w1_ref[k], (((0,), (1,)), ((), ())),
                             preferred_element_type=jnp.float32)
             for k in range(_K)]
    accs = []
    for pp in range(_PP):
        z_stack = jnp.concatenate(
            [z_all[k][pp * _NP:(pp + 1) * _NP] for k in range(_K)],
            axis=0)
        accs.append(lax.dot_general(s_stack, z_stack,
                                    (((0,), (0,)), ((), ())),
                                    preferred_element_type=jnp.float32))
    y_stack = _leaky(jnp.concatenate(accs, axis=0))
    yimg = lax.dot_general(y_stack, e_ref[...], (((0,), (1,)), ((), ())),
                           preferred_element_type=jnp.float32)
    z2 = jnp.concatenate([lf, yimg], axis=0)
    hi = lax.broadcasted_iota(jnp.int32, (2 * _C, _PIX), 1) // _W
    wi = lax.broadcasted_iota(jnp.int32, (2 * _C, _PIX), 1) % _W
    parts = [jnp.zeros((_C, _PIX), dtype=jnp.float32) for _ in range(3)]
    for dy in range(3):
        for dx in range(3):
            t = dy * 3 + dx
            sft = _W * (dy - 1) + (dx - 1)
            mask = None
            if dy == 0:
                mask = hi >= 1
            elif dy == 2:
                mask = hi < (_H - 1)
            if dx == 0:
                mw = wi >= 1
                mask = mw if mask is None else (mask & mw)
            elif dx == 2:
                mw = wi < (_W - 1)
                mask = mw if mask is None else (mask & mw)
            zs = z2 if sft == 0 else pltpu.roll(z2, (-sft) % _PIX, 1)
            if mask is not None:
                zs = jnp.where(mask, zs, 0.0)
            parts[t % 3] = parts[t % 3] + lax.dot_general(
                w2_ref[t], zs, (((1,), (0,)), ((), ())),
                preferred_element_type=jnp.float32)
    o_ref[bi] = _leaky(parts[0] + parts[1] + parts[2])


@jax.jit
def kernel(lf_fea, w_agg1, w_agg2):
    B = lf_fea.shape[0]
    lf2d = lf_fea.reshape(B, _C, _PIX)
    e = jnp.asarray(_E_NP)
    w1 = w_agg1.reshape(_C, _K, _C).transpose(1, 0, 2)
    w2 = w_agg2.transpose(2, 3, 0, 1).reshape(9, _C, 2 * _C)

    dt = pl.pallas_call(
        _dist_kernel,
        grid=(B // _BB,),
        in_specs=[
            pl.BlockSpec((_BB, _C, _PIX), lambda b: (b, 0, 0)),
            pl.BlockSpec((_PIX, _PIX), lambda b: (0, 0)),
        ],
        out_specs=pl.BlockSpec((_BB, _NG, _NP + 8, _GQ),
                               lambda b: (b, 0, 0, 0)),
        out_shape=jax.ShapeDtypeStruct((B, _NG, _NP + 8, _GQ),
                                       jnp.float32),
    )(lf2d, e)

    num_groups = B * _NG
    mesh = plsc.VectorSubcoreMesh(core_axis_name="c", subcore_axis_name="s")
    idx = pl.kernel(
        functools.partial(_sc_body, num_groups),
        out_type=jax.ShapeDtypeStruct((B, _NG, _K, _GQ), jnp.int32),
        mesh=mesh,
        scratch_types=[
            pltpu.VMEM((_NP + 8, _GQ), jnp.float32),
            pltpu.VMEM((_K, _GQ), jnp.int32),
            pltpu.SemaphoreType.DMA,
        ],
    )(dt)

    out = pl.pallas_call(
        _main_kernel,
        grid=(B // _BB,),
        in_specs=[
            pl.BlockSpec((_BB, _C, _PIX), lambda b: (b, 0, 0)),
            pl.BlockSpec((_BB, _NG, _K, _GQ), lambda b: (b, 0, 0, 0)),
            pl.BlockSpec((_PIX, _PIX), lambda b: (0, 0)),
            pl.BlockSpec((_K, _C, _C), lambda b: (0, 0, 0)),
            pl.BlockSpec((9, _C, 2 * _C), lambda b: (0, 0, 0)),
        ],
        out_specs=pl.BlockSpec((_BB, _C, _PIX), lambda b: (b, 0, 0)),
        out_shape=jax.ShapeDtypeStruct((B, _C, _PIX), jnp.float32),
    )(lf2d, idx, e, w1, w2)
    return out.reshape(B, _C, _H, _W)
